# Initial kernel scaffold; baseline (speedup 1.0000x reference)
import jax
import jax.numpy as jnp
from jax.experimental import pallas as pl


def kernel(queries, keys):
    raise NotImplementedError("write your pallas kernel here")


# speedup vs baseline: 1.9545x; 1.9545x over previous
import jax
import jax.numpy as jnp
from jax.experimental import pallas as pl
from jax.experimental.pallas import tpu as pltpu

TOPK = 10
QB = 256
BK = 2048
PAD_VAL = 1e4


def _topk_kernel(q_ref, kt_ref, dv_ref, di_ref, rv_ref, ri_ref):
    ki = pl.program_id(1)
    nk = pl.num_programs(1)

    @pl.when(ki == 0)
    def _init():
        rv_ref[:, :] = jnp.full((QB, 128), jnp.inf, dtype=jnp.float32)
        ri_ref[:, :] = jnp.zeros((QB, 128), dtype=jnp.int32)

    q = q_ref[:, :]
    kt = kt_ref[:, :]
    qsq = jnp.sum(q * q, axis=1, keepdims=True)
    ksq = jnp.sum(kt * kt, axis=0, keepdims=True)
    mm = jnp.dot(q, kt, preferred_element_type=jnp.float32)
    d2 = qsq + ksq - 2.0 * mm

    base = ki * BK
    gidx = base + jax.lax.broadcasted_iota(jnp.int32, (QB, BK), 1)

    wv = jnp.concatenate([rv_ref[:, :], d2], axis=1)
    wi = jnp.concatenate([ri_ref[:, :], gidx], axis=1)

    vals = []
    idxs = []
    for _ in range(TOPK):
        mv = jnp.min(wv, axis=1, keepdims=True)
        eq = wv == mv
        mi = jnp.min(jnp.where(eq, wi, jnp.int32(2**31 - 1)),
                     axis=1, keepdims=True)
        vals.append(mv)
        idxs.append(mi)
        wv = jnp.where(eq & (wi == mi), jnp.inf, wv)
    nv = jnp.concatenate(vals, axis=1)
    ni = jnp.concatenate(idxs, axis=1)
    pad_v = jnp.full((QB, 128 - TOPK), jnp.inf, dtype=jnp.float32)
    pad_i = jnp.zeros((QB, 128 - TOPK), dtype=jnp.int32)
    rv_ref[:, :] = jnp.concatenate([nv, pad_v], axis=1)
    ri_ref[:, :] = jnp.concatenate([ni, pad_i], axis=1)

    @pl.when(ki == nk - 1)
    def _emit():
        dv_ref[:, :] = jnp.sqrt(jnp.maximum(rv_ref[:, :], 0.0) + 1e-12)
        di_ref[:, :] = ri_ref[:, :]


def kernel(queries, keys):
    Q, D = queries.shape
    K = keys.shape[0]
    KP = ((K + BK - 1) // BK) * BK
    keys_t = jnp.pad(keys, ((0, KP - K), (0, 0)),
                     constant_values=PAD_VAL).T
    grid = (Q // QB, KP // BK)
    out_v, out_i = pl.pallas_call(
        _topk_kernel,
        grid=grid,
        in_specs=[
            pl.BlockSpec((QB, D), lambda qi, ki: (qi, 0)),
            pl.BlockSpec((D, BK), lambda qi, ki: (0, ki)),
        ],
        out_specs=[
            pl.BlockSpec((QB, 128), lambda qi, ki: (qi, 0)),
            pl.BlockSpec((QB, 128), lambda qi, ki: (qi, 0)),
        ],
        out_shape=[
            jax.ShapeDtypeStruct((Q, 128), jnp.float32),
            jax.ShapeDtypeStruct((Q, 128), jnp.int32),
        ],
        scratch_shapes=[
            pltpu.VMEM((QB, 128), jnp.float32),
            pltpu.VMEM((QB, 128), jnp.int32),
        ],
    )(queries, keys_t)
    return out_v[:, :TOPK], out_i[:, :TOPK]


# speedup vs baseline: 3.8316x; 1.9604x over previous
import jax
import jax.numpy as jnp
from jax.experimental import pallas as pl
from jax.experimental.pallas import tpu as pltpu

TOPK = 10
QB = 256
BK = 4096
NLANES = 128
DEPTH = 4
PAD_VAL = 1e4
IMAX = 2**31 - 1


def _topk_kernel(q_ref, kt_ref, dv_ref, di_ref, rv_ref, ri_ref):
    ki = pl.program_id(1)
    nk = pl.num_programs(1)

    @pl.when(ki == 0)
    def _init():
        rv_ref[:, :] = jnp.full((QB, NLANES), jnp.inf, dtype=jnp.float32)
        ri_ref[:, :] = jnp.full((QB, NLANES), -1, dtype=jnp.int32)

    q = q_ref[:, :]
    kt = kt_ref[:, :]
    qsq = jnp.sum(q * q, axis=1, keepdims=True)
    ksq = jnp.sum(kt * kt, axis=0, keepdims=True)
    mm = jnp.dot(q, kt, preferred_element_type=jnp.float32)
    d2 = qsq + ksq - 2.0 * mm

    base = ki * BK
    lane = jax.lax.broadcasted_iota(jnp.int32, (QB, NLANES), 1)

    G = BK // NLANES
    mv_ = [jnp.full((QB, NLANES), jnp.inf, dtype=jnp.float32)
           for _ in range(DEPTH)]
    mi_ = [jnp.full((QB, NLANES), -1, dtype=jnp.int32) for _ in range(DEPTH)]
    for g in range(G):
        v = d2[:, g * NLANES:(g + 1) * NLANES]
        vi = lane + (base + g * NLANES)
        for lvl in range(DEPTH):
            c = v < mv_[lvl]
            nm = jnp.where(c, v, mv_[lvl])
            ni = jnp.where(c, vi, mi_[lvl])
            v = jnp.where(c, mv_[lvl], v)
            vi = jnp.where(c, mi_[lvl], vi)
            mv_[lvl] = nm
            mi_[lvl] = ni

    run_v = rv_ref[:, :]
    run_i = ri_ref[:, :]
    orig_rv, orig_ri = run_v, run_i

    popc = jnp.zeros((QB, NLANES), dtype=jnp.int32)
    vals = []
    idxs = []
    for _ in range(TOPK):
        wv = jnp.concatenate([run_v, mv_[0]], axis=1)
        wi = jnp.concatenate([run_i, mi_[0]], axis=1)
        m = jnp.min(wv, axis=1, keepdims=True)
        eq = wv == m
        sel = jnp.min(jnp.where(eq, wi, IMAX), axis=1, keepdims=True)
        vals.append(m)
        idxs.append(sel)
        run_v = jnp.where(run_i == sel, jnp.inf, run_v)
        lm = mi_[0] == sel
        for lvl in range(DEPTH - 1):
            mv_[lvl] = jnp.where(lm, mv_[lvl + 1], mv_[lvl])
            mi_[lvl] = jnp.where(lm, mi_[lvl + 1], mi_[lvl])
        mv_[DEPTH - 1] = jnp.where(lm, jnp.inf, mv_[DEPTH - 1])
        mi_[DEPTH - 1] = jnp.where(lm, -1, mi_[DEPTH - 1])
        popc = popc + lm.astype(jnp.int32)
    nv = jnp.concatenate(vals, axis=1)
    ni = jnp.concatenate(idxs, axis=1)
    pad_v = jnp.full((QB, NLANES - TOPK), jnp.inf, dtype=jnp.float32)
    pad_i = jnp.full((QB, NLANES - TOPK), -1, dtype=jnp.int32)
    rv_ref[:, :] = jnp.concatenate([nv, pad_v], axis=1)
    ri_ref[:, :] = jnp.concatenate([ni, pad_i], axis=1)

    overflow = jnp.max(popc) >= DEPTH

    @pl.when(overflow)
    def _fallback():
        gidx = base + jax.lax.broadcasted_iota(jnp.int32, (QB, BK), 1)
        fwv = jnp.concatenate([orig_rv, d2], axis=1)
        fwi = jnp.concatenate([orig_ri, gidx], axis=1)
        fv = []
        fi = []
        wv = fwv
        for _ in range(TOPK):
            m = jnp.min(wv, axis=1, keepdims=True)
            eq = wv == m
            sel = jnp.min(jnp.where(eq, fwi, IMAX), axis=1, keepdims=True)
            fv.append(m)
            fi.append(sel)
            wv = jnp.where(fwi == sel, jnp.inf, wv)
        rv_ref[:, :] = jnp.concatenate(fv + [pad_v], axis=1)
        ri_ref[:, :] = jnp.concatenate(fi + [pad_i], axis=1)

    @pl.when(ki == nk - 1)
    def _emit():
        dv_ref[:, :] = jnp.sqrt(jnp.maximum(rv_ref[:, :], 0.0) + 1e-12)
        di_ref[:, :] = ri_ref[:, :]


def kernel(queries, keys):
    Q, D = queries.shape
    K = keys.shape[0]
    KP = ((K + BK - 1) // BK) * BK
    keys_t = jnp.pad(keys, ((0, KP - K), (0, 0)),
                     constant_values=PAD_VAL).T
    grid = (Q // QB, KP // BK)
    out_v, out_i = pl.pallas_call(
        _topk_kernel,
        grid=grid,
        in_specs=[
            pl.BlockSpec((QB, D), lambda qi, ki: (qi, 0)),
            pl.BlockSpec((D, BK), lambda qi, ki: (0, ki)),
        ],
        out_specs=[
            pl.BlockSpec((QB, NLANES), lambda qi, ki: (qi, 0)),
            pl.BlockSpec((QB, NLANES), lambda qi, ki: (qi, 0)),
        ],
        out_shape=[
            jax.ShapeDtypeStruct((Q, NLANES), jnp.float32),
            jax.ShapeDtypeStruct((Q, NLANES), jnp.int32),
        ],
        scratch_shapes=[
            pltpu.VMEM((QB, NLANES), jnp.float32),
            pltpu.VMEM((QB, NLANES), jnp.int32),
        ],
    )(queries, keys_t)
    return out_v[:, :TOPK], out_i[:, :TOPK]


# speedup vs baseline: 3.9622x; 1.0341x over previous
import jax
import jax.numpy as jnp
from jax.experimental import pallas as pl
from jax.experimental.pallas import tpu as pltpu

TOPK = 10
QB = 256
BK = 4096
NLANES = 128
DEPTH = 4
PAD_VAL = 1e4
IMAX = 2**31 - 1


def _topk_kernel(q_ref, kt_ref, dv_ref, di_ref, rv_ref, ri_ref):
    ki = pl.program_id(1)
    nk = pl.num_programs(1)

    @pl.when(ki == 0)
    def _init():
        rv_ref[:, :] = jnp.full((QB, NLANES), jnp.inf, dtype=jnp.float32)
        ri_ref[:, :] = jnp.full((QB, NLANES), -1, dtype=jnp.int32)

    q2 = q_ref[:, :]
    kt = kt_ref[:, :]
    ksq = jnp.sum(kt * kt, axis=0, keepdims=True)
    mm2 = jnp.dot(q2, kt, preferred_element_type=jnp.float32)
    d2 = mm2 + ksq

    base = ki * BK
    lane = jax.lax.broadcasted_iota(jnp.int32, (QB, NLANES), 1)

    G = BK // NLANES
    mv_ = [jnp.full((QB, NLANES), jnp.inf, dtype=jnp.float32)
           for _ in range(DEPTH)]
    mi_ = [jnp.full((QB, NLANES), -1, dtype=jnp.int32) for _ in range(DEPTH)]
    for g in range(G):
        v = d2[:, g * NLANES:(g + 1) * NLANES]
        vi = lane + (base + g * NLANES)
        for lvl in range(DEPTH):
            c = v < mv_[lvl]
            nm = jnp.where(c, v, mv_[lvl])
            ni = jnp.where(c, vi, mi_[lvl])
            v = jnp.where(c, mv_[lvl], v)
            vi = jnp.where(c, mi_[lvl], vi)
            mv_[lvl] = nm
            mi_[lvl] = ni

    run_v = rv_ref[:, :]
    run_i = ri_ref[:, :]
    orig_rv, orig_ri = run_v, run_i

    popc = jnp.zeros((QB, NLANES), dtype=jnp.int32)
    vals = []
    idxs = []
    for _ in range(TOPK):
        wv = jnp.concatenate([run_v, mv_[0]], axis=1)
        wi = jnp.concatenate([run_i, mi_[0]], axis=1)
        m = jnp.min(wv, axis=1, keepdims=True)
        eq = wv == m
        sel = jnp.min(jnp.where(eq, wi, IMAX), axis=1, keepdims=True)
        vals.append(m)
        idxs.append(sel)
        run_v = jnp.where(run_i == sel, jnp.inf, run_v)
        lm = mi_[0] == sel
        for lvl in range(DEPTH - 1):
            mv_[lvl] = jnp.where(lm, mv_[lvl + 1], mv_[lvl])
            mi_[lvl] = jnp.where(lm, mi_[lvl + 1], mi_[lvl])
        mv_[DEPTH - 1] = jnp.where(lm, jnp.inf, mv_[DEPTH - 1])
        mi_[DEPTH - 1] = jnp.where(lm, -1, mi_[DEPTH - 1])
        popc = popc + lm.astype(jnp.int32)
    nv = jnp.concatenate(vals, axis=1)
    ni = jnp.concatenate(idxs, axis=1)
    pad_v = jnp.full((QB, NLANES - TOPK), jnp.inf, dtype=jnp.float32)
    pad_i = jnp.full((QB, NLANES - TOPK), -1, dtype=jnp.int32)
    rv_ref[:, :] = jnp.concatenate([nv, pad_v], axis=1)
    ri_ref[:, :] = jnp.concatenate([ni, pad_i], axis=1)

    overflow = jnp.max(popc) >= DEPTH

    @pl.when(overflow)
    def _fallback():
        gidx = base + jax.lax.broadcasted_iota(jnp.int32, (QB, BK), 1)
        fwv = jnp.concatenate([orig_rv, d2], axis=1)
        fwi = jnp.concatenate([orig_ri, gidx], axis=1)
        fv = []
        fi = []
        wv = fwv
        for _ in range(TOPK):
            m = jnp.min(wv, axis=1, keepdims=True)
            eq = wv == m
            sel = jnp.min(jnp.where(eq, fwi, IMAX), axis=1, keepdims=True)
            fv.append(m)
            fi.append(sel)
            wv = jnp.where(fwi == sel, jnp.inf, wv)
        rv_ref[:, :] = jnp.concatenate(fv + [pad_v], axis=1)
        ri_ref[:, :] = jnp.concatenate(fi + [pad_i], axis=1)

    @pl.when(ki == nk - 1)
    def _emit():
        qsq = 0.25 * jnp.sum(q2 * q2, axis=1, keepdims=True)
        dv_ref[:, :] = jnp.sqrt(
            jnp.maximum(rv_ref[:, :] + qsq, 0.0) + 1e-12)
        di_ref[:, :] = ri_ref[:, :]


def kernel(queries, keys):
    Q, D = queries.shape
    K = keys.shape[0]
    KP = ((K + BK - 1) // BK) * BK
    keys_t = jnp.pad(keys, ((0, KP - K), (0, 0)),
                     constant_values=PAD_VAL).T
    queries2 = queries * (-2.0)
    grid = (Q // QB, KP // BK)
    out_v, out_i = pl.pallas_call(
        _topk_kernel,
        grid=grid,
        in_specs=[
            pl.BlockSpec((QB, D), lambda qi, ki: (qi, 0)),
            pl.BlockSpec((D, BK), lambda qi, ki: (0, ki)),
        ],
        out_specs=[
            pl.BlockSpec((QB, NLANES), lambda qi, ki: (qi, 0)),
            pl.BlockSpec((QB, NLANES), lambda qi, ki: (qi, 0)),
        ],
        out_shape=[
            jax.ShapeDtypeStruct((Q, NLANES), jnp.float32),
            jax.ShapeDtypeStruct((Q, NLANES), jnp.int32),
        ],
        scratch_shapes=[
            pltpu.VMEM((QB, NLANES), jnp.float32),
            pltpu.VMEM((QB, NLANES), jnp.int32),
        ],
    )(queries2, keys_t)
    return out_v[:, :TOPK], out_i[:, :TOPK]


# speedup vs baseline: 4.3361x; 1.0944x over previous
import jax
import jax.numpy as jnp
from jax.experimental import pallas as pl
from jax.experimental.pallas import tpu as pltpu

TOPK = 10
QB = 256
BK = 4096
NLANES = 128
DEPTH = 3
PAD_VAL = 1e4
IMAX = 2**31 - 1


def _topk_kernel(q_ref, kt_ref, dv_ref, di_ref, rv_ref, ri_ref):
    ki = pl.program_id(1)
    nk = pl.num_programs(1)

    @pl.when(ki == 0)
    def _init():
        rv_ref[:, :] = jnp.full((QB, NLANES), jnp.inf, dtype=jnp.float32)
        ri_ref[:, :] = jnp.full((QB, NLANES), -1, dtype=jnp.int32)

    q2 = q_ref[:, :]
    kt = kt_ref[:, :]
    ksq = jnp.sum(kt * kt, axis=0, keepdims=True)
    mm2 = jnp.dot(q2, kt, preferred_element_type=jnp.float32)
    d2 = mm2 + ksq

    base = ki * BK
    lane = jax.lax.broadcasted_iota(jnp.int32, (QB, NLANES), 1)

    G = BK // NLANES
    mv_ = [jnp.full((QB, NLANES), jnp.inf, dtype=jnp.float32)
           for _ in range(DEPTH)]
    mi_ = [jnp.full((QB, NLANES), -1, dtype=jnp.int32) for _ in range(DEPTH)]
    m4 = jnp.full((QB, NLANES), jnp.inf, dtype=jnp.float32)
    for g in range(G):
        v = d2[:, g * NLANES:(g + 1) * NLANES]
        vi = lane + (base + g * NLANES)
        for lvl in range(DEPTH):
            c = v < mv_[lvl]
            nm = jnp.where(c, v, mv_[lvl])
            ni = jnp.where(c, vi, mi_[lvl])
            v = jnp.where(c, mv_[lvl], v)
            vi = jnp.where(c, mi_[lvl], vi)
            mv_[lvl] = nm
            mi_[lvl] = ni
        m4 = jnp.minimum(m4, v)

    run_v = rv_ref[:, :]
    run_i = ri_ref[:, :]
    orig_rv, orig_ri = run_v, run_i

    popc = jnp.zeros((QB, NLANES), dtype=jnp.int32)
    vals = []
    idxs = []
    for _ in range(TOPK):
        wv = jnp.concatenate([run_v, mv_[0]], axis=1)
        wi = jnp.concatenate([run_i, mi_[0]], axis=1)
        m = jnp.min(wv, axis=1, keepdims=True)
        eq = wv == m
        sel = jnp.min(jnp.where(eq, wi, IMAX), axis=1, keepdims=True)
        vals.append(m)
        idxs.append(sel)
        run_v = jnp.where(run_i == sel, jnp.inf, run_v)
        lm = mi_[0] == sel
        for lvl in range(DEPTH - 1):
            mv_[lvl] = jnp.where(lm, mv_[lvl + 1], mv_[lvl])
            mi_[lvl] = jnp.where(lm, mi_[lvl + 1], mi_[lvl])
        mv_[DEPTH - 1] = jnp.where(lm, jnp.inf, mv_[DEPTH - 1])
        mi_[DEPTH - 1] = jnp.where(lm, -1, mi_[DEPTH - 1])
        popc = popc + lm.astype(jnp.int32)
    nv = jnp.concatenate(vals, axis=1)
    ni = jnp.concatenate(idxs, axis=1)
    pad_v = jnp.full((QB, NLANES - TOPK), jnp.inf, dtype=jnp.float32)
    pad_i = jnp.full((QB, NLANES - TOPK), -1, dtype=jnp.int32)
    rv_ref[:, :] = jnp.concatenate([nv, pad_v], axis=1)
    ri_ref[:, :] = jnp.concatenate([ni, pad_i], axis=1)

    tau = vals[TOPK - 1]
    overflow = jnp.any((popc >= DEPTH) & (m4 <= tau))

    @pl.when(overflow)
    def _fallback():
        gidx = base + jax.lax.broadcasted_iota(jnp.int32, (QB, BK), 1)
        fwv = jnp.concatenate([orig_rv, d2], axis=1)
        fwi = jnp.concatenate([orig_ri, gidx], axis=1)
        fv = []
        fi = []
        wv = fwv
        for _ in range(TOPK):
            m = jnp.min(wv, axis=1, keepdims=True)
            eq = wv == m
            sel = jnp.min(jnp.where(eq, fwi, IMAX), axis=1, keepdims=True)
            fv.append(m)
            fi.append(sel)
            wv = jnp.where(fwi == sel, jnp.inf, wv)
        rv_ref[:, :] = jnp.concatenate(fv + [pad_v], axis=1)
        ri_ref[:, :] = jnp.concatenate(fi + [pad_i], axis=1)

    @pl.when(ki == nk - 1)
    def _emit():
        qsq = 0.25 * jnp.sum(q2 * q2, axis=1, keepdims=True)
        dv_ref[:, :] = jnp.sqrt(
            jnp.maximum(rv_ref[:, :] + qsq, 0.0) + 1e-12)
        di_ref[:, :] = ri_ref[:, :]


def kernel(queries, keys):
    Q, D = queries.shape
    K = keys.shape[0]
    KP = ((K + BK - 1) // BK) * BK
    keys_t = jnp.pad(keys, ((0, KP - K), (0, 0)),
                     constant_values=PAD_VAL).T
    queries2 = queries * (-2.0)
    grid = (Q // QB, KP // BK)
    out_v, out_i = pl.pallas_call(
        _topk_kernel,
        grid=grid,
        in_specs=[
            pl.BlockSpec((QB, D), lambda qi, ki: (qi, 0)),
            pl.BlockSpec((D, BK), lambda qi, ki: (0, ki)),
        ],
        out_specs=[
            pl.BlockSpec((QB, NLANES), lambda qi, ki: (qi, 0)),
            pl.BlockSpec((QB, NLANES), lambda qi, ki: (qi, 0)),
        ],
        out_shape=[
            jax.ShapeDtypeStruct((Q, NLANES), jnp.float32),
            jax.ShapeDtypeStruct((Q, NLANES), jnp.int32),
        ],
        scratch_shapes=[
            pltpu.VMEM((QB, NLANES), jnp.float32),
            pltpu.VMEM((QB, NLANES), jnp.int32),
        ],
    )(queries2, keys_t)
    return out_v[:, :TOPK], out_i[:, :TOPK]


# speedup vs baseline: 5.1150x; 1.1796x over previous
import jax
import jax.numpy as jnp
from jax.experimental import pallas as pl
from jax.experimental.pallas import tpu as pltpu

TOPK = 10
QB = 256
BK = 8192
NLANES = 128
DEPTH = 3
PAD_VAL = 1e4
IMAX = 2**31 - 1


def _topk_kernel(q_ref, kt_ref, dv_ref, di_ref, rv_ref, ri_ref):
    ki = pl.program_id(1)
    nk = pl.num_programs(1)

    @pl.when(ki == 0)
    def _init():
        rv_ref[:, :] = jnp.full((QB, NLANES), jnp.inf, dtype=jnp.float32)
        ri_ref[:, :] = jnp.full((QB, NLANES), -1, dtype=jnp.int32)

    q2 = q_ref[:, :]
    kt = kt_ref[:, :]
    ksq = jnp.sum(kt * kt, axis=0, keepdims=True)
    mm2 = jnp.dot(q2, kt, preferred_element_type=jnp.float32)
    d2 = mm2 + ksq

    base = ki * BK
    lane = jax.lax.broadcasted_iota(jnp.int32, (QB, NLANES), 1)

    G = BK // NLANES
    mv_ = [jnp.full((QB, NLANES), jnp.inf, dtype=jnp.float32)
           for _ in range(DEPTH)]
    mi_ = [jnp.full((QB, NLANES), -1, dtype=jnp.int32) for _ in range(DEPTH)]
    m4 = jnp.full((QB, NLANES), jnp.inf, dtype=jnp.float32)
    for g in range(G):
        v = d2[:, g * NLANES:(g + 1) * NLANES]
        vi = lane + (base + g * NLANES)
        for lvl in range(DEPTH):
            c = v < mv_[lvl]
            nm = jnp.where(c, v, mv_[lvl])
            ni = jnp.where(c, vi, mi_[lvl])
            v = jnp.where(c, mv_[lvl], v)
            vi = jnp.where(c, mi_[lvl], vi)
            mv_[lvl] = nm
            mi_[lvl] = ni
        m4 = jnp.minimum(m4, v)

    run_v = rv_ref[:, :]
    run_i = ri_ref[:, :]
    orig_rv, orig_ri = run_v, run_i

    popc = jnp.zeros((QB, NLANES), dtype=jnp.int32)
    vals = []
    idxs = []
    for _ in range(TOPK):
        wv = jnp.concatenate([run_v, mv_[0]], axis=1)
        wi = jnp.concatenate([run_i, mi_[0]], axis=1)
        m = jnp.min(wv, axis=1, keepdims=True)
        eq = wv == m
        sel = jnp.min(jnp.where(eq, wi, IMAX), axis=1, keepdims=True)
        vals.append(m)
        idxs.append(sel)
        run_v = jnp.where(run_i == sel, jnp.inf, run_v)
        lm = mi_[0] == sel
        for lvl in range(DEPTH - 1):
            mv_[lvl] = jnp.where(lm, mv_[lvl + 1], mv_[lvl])
            mi_[lvl] = jnp.where(lm, mi_[lvl + 1], mi_[lvl])
        mv_[DEPTH - 1] = jnp.where(lm, jnp.inf, mv_[DEPTH - 1])
        mi_[DEPTH - 1] = jnp.where(lm, -1, mi_[DEPTH - 1])
        popc = popc + lm.astype(jnp.int32)
    nv = jnp.concatenate(vals, axis=1)
    ni = jnp.concatenate(idxs, axis=1)
    pad_v = jnp.full((QB, NLANES - TOPK), jnp.inf, dtype=jnp.float32)
    pad_i = jnp.full((QB, NLANES - TOPK), -1, dtype=jnp.int32)
    rv_ref[:, :] = jnp.concatenate([nv, pad_v], axis=1)
    ri_ref[:, :] = jnp.concatenate([ni, pad_i], axis=1)

    tau = vals[TOPK - 1]
    overflow = jnp.any((popc >= DEPTH) & (m4 <= tau))

    @pl.when(overflow)
    def _fallback():
        gidx = base + jax.lax.broadcasted_iota(jnp.int32, (QB, BK), 1)
        fwv = jnp.concatenate([orig_rv, d2], axis=1)
        fwi = jnp.concatenate([orig_ri, gidx], axis=1)
        fv = []
        fi = []
        wv = fwv
        for _ in range(TOPK):
            m = jnp.min(wv, axis=1, keepdims=True)
            eq = wv == m
            sel = jnp.min(jnp.where(eq, fwi, IMAX), axis=1, keepdims=True)
            fv.append(m)
            fi.append(sel)
            wv = jnp.where(fwi == sel, jnp.inf, wv)
        rv_ref[:, :] = jnp.concatenate(fv + [pad_v], axis=1)
        ri_ref[:, :] = jnp.concatenate(fi + [pad_i], axis=1)

    @pl.when(ki == nk - 1)
    def _emit():
        qsq = 0.25 * jnp.sum(q2 * q2, axis=1, keepdims=True)
        dv_ref[:, :] = jnp.sqrt(
            jnp.maximum(rv_ref[:, :] + qsq, 0.0) + 1e-12)
        di_ref[:, :] = ri_ref[:, :]


def kernel(queries, keys):
    Q, D = queries.shape
    K = keys.shape[0]
    KP = ((K + BK - 1) // BK) * BK
    keys_t = jnp.pad(keys, ((0, KP - K), (0, 0)),
                     constant_values=PAD_VAL).T
    queries2 = queries * (-2.0)
    grid = (Q // QB, KP // BK)
    out_v, out_i = pl.pallas_call(
        _topk_kernel,
        grid=grid,
        in_specs=[
            pl.BlockSpec((QB, D), lambda qi, ki: (qi, 0)),
            pl.BlockSpec((D, BK), lambda qi, ki: (0, ki)),
        ],
        out_specs=[
            pl.BlockSpec((QB, NLANES), lambda qi, ki: (qi, 0)),
            pl.BlockSpec((QB, NLANES), lambda qi, ki: (qi, 0)),
        ],
        out_shape=[
            jax.ShapeDtypeStruct((Q, NLANES), jnp.float32),
            jax.ShapeDtypeStruct((Q, NLANES), jnp.int32),
        ],
        scratch_shapes=[
            pltpu.VMEM((QB, NLANES), jnp.float32),
            pltpu.VMEM((QB, NLANES), jnp.int32),
        ],
    )(queries2, keys_t)
    return out_v[:, :TOPK], out_i[:, :TOPK]
